# Initial kernel scaffold; baseline (speedup 1.0000x reference)
import jax
import jax.numpy as jnp
from jax.experimental import pallas as pl


def kernel(indices, table):
    raise NotImplementedError("write your pallas kernel here")


# speedup vs baseline: 1.0820x; 1.0820x over previous
import functools

import jax
import jax.numpy as jnp
from jax import lax
from jax.experimental import pallas as pl
from jax.experimental.pallas import tpu as pltpu
from jax.experimental.pallas import tpu_sc as plsc

_B, _L = 4096, 200
_N = _B * _L
_D = 32
_NC, _NS = 2, 16
_NW = _NC * _NS
_PER_W = _N // _NW
_IDXROW = 128
_CPB = 10
_CHUNK = _CPB * _IDXROW
_NCHUNK = _PER_W // _CHUNK
_NBUF = 2

_mesh = plsc.VectorSubcoreMesh(core_axis_name="c", subcore_axis_name="s")


@functools.partial(
    pl.kernel,
    mesh=_mesh,
    compiler_params=pltpu.CompilerParams(use_tc_tiling_on_sc=False),
    out_type=jax.ShapeDtypeStruct((_NW, _NCHUNK, _CHUNK, _D), jnp.float32),
    scratch_types=[
        pltpu.VMEM((_NBUF, _CPB, _IDXROW), jnp.int32),
        pltpu.VMEM((_NBUF, _CHUNK, _D), jnp.float32),
        pltpu.SemaphoreType.DMA((_NBUF,)),
        pltpu.SemaphoreType.DMA((_NBUF,)),
        pltpu.SemaphoreType.DMA((_NBUF,)),
    ],
)
def _gather_rows(idx_hbm, table_hbm, out_hbm, idx_v, rows_v, isem, gsem, osem):
    wid = lax.axis_index("s") * _NC + lax.axis_index("c")

    def idx_copy(g, slot):
        return pltpu.make_async_copy(
            idx_hbm.at[wid, g], idx_v.at[slot], isem.at[slot])

    def out_copy(g, slot):
        return pltpu.make_async_copy(
            rows_v.at[slot], out_hbm.at[wid, g], osem.at[slot])

    def run_gathers(slot):
        handles = [
            pltpu.make_async_copy(
                table_hbm.at[idx_v.at[slot, j]],
                rows_v.at[slot, pl.ds(j * _IDXROW, _IDXROW)],
                gsem.at[slot],
            )
            for j in range(_CPB)
        ]
        for h in handles:
            h.start()
        for h in handles:
            h.wait()

    idx_copy(0, 0).start()
    idx_copy(1, 1).start()

    for b in range(_NBUF):
        idx_copy(b, b).wait()
        run_gathers(b)
        idx_copy(b + _NBUF, b).start()
        out_copy(b, b).start()

    def ring(i, carry):
        for b in range(_NBUF):
            g = _NBUF * i + b
            idx_copy(g, b).wait()
            out_copy(g - _NBUF, b).wait()
            run_gathers(b)
            idx_copy(g + _NBUF, b).start()
            out_copy(g, b).start()
        return carry

    lax.fori_loop(1, _NCHUNK // _NBUF - 1, ring, 0)

    for b in range(_NBUF):
        g = _NCHUNK - _NBUF + b
        idx_copy(g, b).wait()
        out_copy(g - _NBUF, b).wait()
        run_gathers(b)
        out_copy(g, b).start()
    for b in range(_NBUF):
        out_copy(_NCHUNK - _NBUF + b, b).wait()


def kernel(indices, table):
    idx4 = indices.reshape(_NW, _NCHUNK, _CPB, _IDXROW)
    rows = _gather_rows(idx4, table)
    all_embs = rows.reshape(_N, 1, _D)
    boundaries = jnp.arange(_B + 1, dtype=jnp.int32) * jnp.int32(_L)
    return (all_embs, boundaries)
